# Initial kernel scaffold; baseline (speedup 1.0000x reference)
import jax
import jax.numpy as jnp
from jax.experimental import pallas as pl


def kernel(x, condition_global, pv_w, pv_b, pq_w, pq_b, pk_w, pk_b, cs1_w, cs1_b, cs2_w, cs2_b, po_w, po_b, rin_w, rin_b, rln_g, rln_b, rm1_w, rm1_b, rm2_w, rm2_b, rca_w, rca_b, ls_w, ls_b):
    raise NotImplementedError("write your pallas kernel here")


# speedup vs baseline: 6.8872x; 6.8872x over previous
import jax
import jax.numpy as jnp
from jax.experimental import pallas as pl
from jax.experimental.pallas import tpu as pltpu

_DIM = 96
_WS = 8
_H = 384
_W = 384
_HH = _H // _WS
_WW = _W // _WS
_NW = _HH * _WW
_WIN = _WS * _WS
_CQ = (_DIM + 4) // 4
_NPIX = _H * _W

_WBLK = 48
_R = 16

_M = int(_NW * 2 * 0.5)
_mm = _M
while _mm % 2 == 0:
    _mm //= 2
_SIGMA_C = float(2 ** _mm.bit_length() + 1)
_MF = float(_M)


def _lrelu(x):
    return jnp.where(x >= 0, x, 0.1 * x)


def _stage1_kernel(xw_ref, cg_ref, cwt_ref,
                   pv_w_ref, pv_b_ref, pq_w_ref, pq_b_ref, pk_w_ref, pk_b_ref,
                   rin_w_ref, rin_b_ref, rln_g_ref, rln_b_ref,
                   vw_ref, fw_ref, tw_ref, tsum_ref):
    nb = _WBLK
    x2 = xw_ref[...].reshape(nb * _WIN, _DIM)
    vv = jnp.dot(x2, pv_w_ref[...].T, preferred_element_type=jnp.float32) + pv_b_ref[...]
    qq = jnp.dot(x2, pq_w_ref[...].T, preferred_element_type=jnp.float32) + pq_b_ref[...]
    kk = jnp.dot(x2, pk_w_ref[...].T, preferred_element_type=jnp.float32) + pk_b_ref[...]

    cg2 = cg_ref[...].reshape(nb * _WIN, 2)
    cw2 = jnp.broadcast_to(cwt_ref[...][None], (nb, _WIN, 2)).reshape(nb * _WIN, 2)
    cond = jnp.concatenate([vv, cg2, cw2], axis=1)
    tt = jnp.dot(cond, rin_w_ref[...].T, preferred_element_type=jnp.float32) + rin_b_ref[...]
    m = jnp.mean(tt, axis=1, keepdims=True)
    var = jnp.mean((tt - m) ** 2, axis=1, keepdims=True)
    tt = (tt - m) / jnp.sqrt(var + 1e-5) * rln_g_ref[...] + rln_b_ref[...]
    tt = _lrelu(tt)

    @pl.when(pl.program_id(0) == 0)
    def _():
        tsum_ref[...] = jnp.zeros_like(tsum_ref)
    tsum_ref[...] += jnp.sum(tt, axis=0, keepdims=True)

    tw_ref[...] = jnp.mean(tt.reshape(nb, _WIN, _CQ), axis=2)

    q3 = qq.reshape(nb, _WIN, _DIM)
    k3 = kk.reshape(nb, _WIN, _DIM)
    v3 = vv.reshape(nb, _WIN, _DIM)
    logits = jax.lax.dot_general(q3, k3, (((2,), (2,)), ((0,), (0,))),
                                 preferred_element_type=jnp.float32)
    a = jax.nn.softmax(logits, axis=-1)
    f3 = jax.lax.dot_general(a, v3, (((2,), (1,)), ((0,), (0,))),
                             preferred_element_type=jnp.float32)
    vw_ref[...] = v3
    fw_ref[...] = f3


def _router_kernel(tw_ref, u_ref, tsum_ref,
                   rm1_w_ref, rm1_b_ref, rm2_w_ref, rm2_b_ref,
                   rca_w_ref, rca_b_ref,
                   keep_ref, ca_ref, score_sc):
    tw = tw_ref[...]
    h1 = _lrelu(jnp.dot(tw, rm1_w_ref[...].T, preferred_element_type=jnp.float32)
                + rm1_b_ref[...])
    lg = jnp.dot(h1, rm2_w_ref[...].T, preferred_element_type=jnp.float32) + rm2_b_ref[...]
    pred = jax.nn.softmax(lg, axis=-1)

    g = pred - jnp.log(-jnp.log(u_ref[...]))
    hard = (g[:, 0:1] >= g[:, 1:2]).astype(jnp.float32)
    kcnt = jnp.sum(hard, axis=0, keepdims=True)
    r = kcnt / float(_NW)
    ts = r * _SIGMA_C
    r_hi = ts - (ts - r)
    r_lo = r - r_hi
    diff = (_MF * r_hi - kcnt) + _MF * r_lo
    nk = jnp.clip(kcnt - (diff < 0).astype(jnp.float32), 1.0, float(_NW))

    score = pred[:, 0:1]
    score_sc[...] = score
    st = jnp.transpose(score)
    jl = jax.lax.broadcasted_iota(jnp.int32, (1, _NW), 1)

    def body(ci, carry):
        base = ci * 128
        sc = score_sc[pl.ds(base, 128), :]
        ir = jax.lax.broadcasted_iota(jnp.int32, (128, 1), 0) + base
        gt = st > sc
        tie = (st == sc) & (jl < ir)
        cnt = jnp.sum((gt | tie).astype(jnp.float32), axis=1, keepdims=True)
        keep_ref[pl.ds(base, 128), :] = (cnt < nk).astype(jnp.float32)
        return carry

    jax.lax.fori_loop(0, _NW // 128, body, 0)

    tmean = tsum_ref[...] * (1.0 / _NPIX)
    ca_ref[...] = jax.nn.sigmoid(
        jnp.dot(tmean, rca_w_ref[...].T, preferred_element_type=jnp.float32)
        + rca_b_ref[...])


def _shift_cols(x, s):
    if s == 0:
        return x
    r, c, ch = x.shape
    z = jnp.zeros((r, abs(s), ch), x.dtype)
    if s > 0:
        return jnp.concatenate([x[:, s:, :], z], axis=1)
    return jnp.concatenate([z, x[:, :s, :]], axis=1)


def _stage2_kernel(vp_ref, vc_ref, vn_ref, fp_ref, fc_ref, fn_ref,
                   keep_ref, exp_ref, ca_ref,
                   bdg_ref, ls_b_ref, w1_ref, b1_ref, w2_ref, b2_ref,
                   po_w_ref, po_b_ref, out_ref):
    i = pl.program_id(0)
    mtop = (i > 0).astype(jnp.float32)
    mbot = (i < (_H // _R) - 1).astype(jnp.float32)
    v_ext = jnp.concatenate(
        [vp_ref[...] * mtop, vc_ref[...], vn_ref[...] * mbot], axis=0)
    f_ext = jnp.concatenate(
        [fp_ref[...] * mtop, fc_ref[...], fn_ref[...] * mbot], axis=0)[5:5 + _R + 6]

    ne = _R + 6
    acc = None
    for t in range(9):
        dy, dx = t // 3 - 1, t % 3 - 1
        sl = _shift_cols(v_ext[5 + dy:5 + ne + dy], dx).reshape(ne * _W, _DIM)
        p = jnp.dot(sl, bdg_ref[t], preferred_element_type=jnp.float32)
        acc = p if acc is None else acc + p
    vs = (acc + ls_b_ref[...]).reshape(ne, _W, _DIM)

    gr = jax.lax.broadcasted_iota(jnp.int32, (ne, 1), 0) + (_R * i - 3)
    wr = jnp.clip(gr // _WS, 0, _HH - 1)
    oh = (jax.lax.broadcasted_iota(jnp.int32, (ne, _HH), 1) == wr).astype(jnp.float32)
    krows = jnp.dot(oh, keep_ref[...], preferred_element_type=jnp.float32)
    kpix = jnp.dot(krows, exp_ref[...], preferred_element_type=jnp.float32)
    rmask = ((gr >= 0) & (gr < _H)).astype(jnp.float32)
    ao = jnp.where(kpix[:, :, None] > 0.5, f_ext, vs) * rmask[:, :, None]

    nd = _R + 4
    d1 = None
    for t in range(9):
        dy, dx = t // 3 - 1, t % 3 - 1
        sl = _shift_cols(ao[1 + dy:1 + nd + dy], dx) * w1_ref[t:t + 1, :][None]
        d1 = sl if d1 is None else d1 + sl
    d1 = d1 + b1_ref[...][None]

    sp = None
    for t in range(9):
        dy, dx = (t // 3 - 1) * 2, (t % 3 - 1) * 2
        sl = _shift_cols(d1[2 + dy:2 + _R + dy], dx) * w2_ref[t:t + 1, :][None]
        sp = sl if sp is None else sp + sl
    sp = sp + b2_ref[...][None]

    ge = 0.5 * sp * (1.0 + jax.lax.erf(sp * (2.0 ** -0.5)))
    outp = ge * ca_ref[...][None] + ao[3:3 + _R]
    fin = (jnp.dot(outp.reshape(_R * _W, _DIM), po_w_ref[...].T,
                   preferred_element_type=jnp.float32) + po_b_ref[...])
    out_ref[...] = fin.reshape(_R, _W, _DIM)


def _row2(v):
    return v.reshape(1, -1)


def kernel(x, condition_global, pv_w, pv_b, pq_w, pq_b, pk_w, pk_b,
           cs1_w, cs1_b, cs2_w, cs2_b, po_w, po_b, rin_w, rin_b,
           rln_g, rln_b, rm1_w, rm1_b, rm2_w, rm2_b, rca_w, rca_b,
           ls_w, ls_b):
    f32 = jnp.float32

    xw = (x[0].reshape(_DIM, _HH, _WS, _WW, _WS)
          .transpose(1, 3, 2, 4, 0).reshape(_NW, _WIN, _DIM))
    cgw = (condition_global[0].reshape(2, _HH, _WS, _WW, _WS)
           .transpose(1, 3, 2, 4, 0).reshape(_NW, _WIN, 2))

    lin = jnp.linspace(-1.0, 1.0, _WS)
    gy, gx = jnp.meshgrid(lin, lin, indexing='ij')
    cwt = jnp.stack([gy, gx], axis=-1).reshape(_WIN, 2).astype(f32)

    u = jax.random.uniform(jax.random.key(42), (1, _NW, 2),
                           minval=1e-6, maxval=1.0 - 1e-6)[0]

    g1 = _H // _R

    sds = jax.ShapeDtypeStruct
    vw, fw, twin, tsum = pl.pallas_call(
        _stage1_kernel,
        grid=(_NW // _WBLK,),
        in_specs=[
            pl.BlockSpec((_WBLK, _WIN, _DIM), lambda i: (i, 0, 0)),
            pl.BlockSpec((_WBLK, _WIN, 2), lambda i: (i, 0, 0)),
            pl.BlockSpec((_WIN, 2), lambda i: (0, 0)),
            pl.BlockSpec((_DIM, _DIM), lambda i: (0, 0)),
            pl.BlockSpec((1, _DIM), lambda i: (0, 0)),
            pl.BlockSpec((_DIM, _DIM), lambda i: (0, 0)),
            pl.BlockSpec((1, _DIM), lambda i: (0, 0)),
            pl.BlockSpec((_DIM, _DIM), lambda i: (0, 0)),
            pl.BlockSpec((1, _DIM), lambda i: (0, 0)),
            pl.BlockSpec((_CQ, _DIM + 4), lambda i: (0, 0)),
            pl.BlockSpec((1, _CQ), lambda i: (0, 0)),
            pl.BlockSpec((1, _CQ), lambda i: (0, 0)),
            pl.BlockSpec((1, _CQ), lambda i: (0, 0)),
        ],
        out_specs=[
            pl.BlockSpec((_WBLK, _WIN, _DIM), lambda i: (i, 0, 0)),
            pl.BlockSpec((_WBLK, _WIN, _DIM), lambda i: (i, 0, 0)),
            pl.BlockSpec((_WBLK, _WIN), lambda i: (i, 0)),
            pl.BlockSpec((1, _CQ), lambda i: (0, 0)),
        ],
        out_shape=[
            sds((_NW, _WIN, _DIM), f32),
            sds((_NW, _WIN, _DIM), f32),
            sds((_NW, _WIN), f32),
            sds((1, _CQ), f32),
        ],
    )(xw, cgw, cwt, pv_w, _row2(pv_b), pq_w, _row2(pq_b), pk_w, _row2(pk_b),
      rin_w, _row2(rin_b), _row2(rln_g), _row2(rln_b))

    keep, ca = pl.pallas_call(
        _router_kernel,
        out_shape=[sds((_NW, 1), f32), sds((1, _DIM), f32)],
        scratch_shapes=[pltpu.VMEM((_NW, 1), f32)],
    )(twin, u, tsum, rm1_w, _row2(rm1_b), rm2_w, _row2(rm2_b),
      rca_w, _row2(rca_b))

    vsp = (vw.reshape(_HH, _WW, _WS, _WS, _DIM)
           .transpose(0, 2, 1, 3, 4).reshape(_H, _W, _DIM))
    fsp = (fw.reshape(_HH, _WW, _WS, _WS, _DIM)
           .transpose(0, 2, 1, 3, 4).reshape(_H, _W, _DIM))
    keep2d = keep.reshape(_HH, _WW)

    grp = _DIM // 6
    ls9 = jnp.tile(ls_w.transpose(2, 3, 1, 0).reshape(9, grp, _DIM), (1, 6, 1))
    ci = jax.lax.broadcasted_iota(jnp.int32, (_DIM, _DIM), 0) // grp
    co = jax.lax.broadcasted_iota(jnp.int32, (_DIM, _DIM), 1) // grp
    bdg = ls9 * (ci == co).astype(f32)[None]

    pc = jax.lax.broadcasted_iota(jnp.int32, (_WW, _W), 1) // _WS
    wc = jax.lax.broadcasted_iota(jnp.int32, (_WW, _W), 0)
    expm = (pc == wc).astype(f32)

    w1 = cs1_w.reshape(_DIM, 9).T
    w2 = cs2_w.reshape(_DIM, 9).T

    out_sp = pl.pallas_call(
        _stage2_kernel,
        grid=(g1,),
        in_specs=[
            pl.BlockSpec((8, _W, _DIM), lambda i: (jnp.maximum(2 * i - 1, 0), 0, 0)),
            pl.BlockSpec((_R, _W, _DIM), lambda i: (i, 0, 0)),
            pl.BlockSpec((8, _W, _DIM), lambda i: (jnp.minimum(2 * i + 2, _H // 8 - 1), 0, 0)),
            pl.BlockSpec((8, _W, _DIM), lambda i: (jnp.maximum(2 * i - 1, 0), 0, 0)),
            pl.BlockSpec((_R, _W, _DIM), lambda i: (i, 0, 0)),
            pl.BlockSpec((8, _W, _DIM), lambda i: (jnp.minimum(2 * i + 2, _H // 8 - 1), 0, 0)),
            pl.BlockSpec((_HH, _WW), lambda i: (0, 0)),
            pl.BlockSpec((_WW, _W), lambda i: (0, 0)),
            pl.BlockSpec((1, _DIM), lambda i: (0, 0)),
            pl.BlockSpec((9, _DIM, _DIM), lambda i: (0, 0, 0)),
            pl.BlockSpec((1, _DIM), lambda i: (0, 0)),
            pl.BlockSpec((9, _DIM), lambda i: (0, 0)),
            pl.BlockSpec((1, _DIM), lambda i: (0, 0)),
            pl.BlockSpec((9, _DIM), lambda i: (0, 0)),
            pl.BlockSpec((1, _DIM), lambda i: (0, 0)),
            pl.BlockSpec((_DIM, _DIM), lambda i: (0, 0)),
            pl.BlockSpec((1, _DIM), lambda i: (0, 0)),
        ],
        out_specs=[pl.BlockSpec((_R, _W, _DIM), lambda i: (i, 0, 0))],
        out_shape=[sds((_H, _W, _DIM), f32)],
    )(vsp, vsp, vsp, fsp, fsp, fsp, keep2d, expm, ca,
      bdg, _row2(ls_b), w1, _row2(cs1_b), w2, _row2(cs2_b),
      po_w, _row2(po_b))[0]

    return out_sp.transpose(2, 0, 1)[None]


# speedup vs baseline: 9.2081x; 1.3370x over previous
import jax
import jax.numpy as jnp
from jax.experimental import pallas as pl
from jax.experimental.pallas import tpu as pltpu

_DIM = 96
_WS = 8
_H = 384
_W = 384
_HH = _H // _WS
_WW = _W // _WS
_NW = _HH * _WW
_WIN = _WS * _WS
_CQ = (_DIM + 4) // 4
_NPIX = _H * _W

_WBLK = 48
_R = 16

_M = int(_NW * 2 * 0.5)
_mm = _M
while _mm % 2 == 0:
    _mm //= 2
_SIGMA_C = float(2 ** _mm.bit_length() + 1)
_MF = float(_M)


def _lrelu(x):
    return jnp.where(x >= 0, x, 0.1 * x)


def _stage1_kernel(xh_ref, cg_ref, cwt_ref,
                   pv_w_ref, pv_b_ref, pq_w_ref, pq_b_ref, pk_w_ref, pk_b_ref,
                   rin_w_ref, rin_b_ref, rln_g_ref, rln_b_ref,
                   vw_ref, fw_ref, tw_ref, tsum_ref):
    nb = _WBLK
    xw = (xh_ref[...].reshape(_WS, _WW, _WS, _DIM)
          .transpose(1, 0, 2, 3).reshape(nb, _WIN, _DIM))
    x2 = xw.reshape(nb * _WIN, _DIM)
    vv = jnp.dot(x2, pv_w_ref[...].T, preferred_element_type=jnp.float32) + pv_b_ref[...]
    qq = jnp.dot(x2, pq_w_ref[...].T, preferred_element_type=jnp.float32) + pq_b_ref[...]
    kk = jnp.dot(x2, pk_w_ref[...].T, preferred_element_type=jnp.float32) + pk_b_ref[...]

    cg2 = (cg_ref[...].reshape(_WS, _WW, _WS, 2)
           .transpose(1, 0, 2, 3).reshape(nb * _WIN, 2))
    cw2 = jnp.broadcast_to(cwt_ref[...][None], (nb, _WIN, 2)).reshape(nb * _WIN, 2)
    cond = jnp.concatenate([vv, cg2, cw2], axis=1)
    tt = jnp.dot(cond, rin_w_ref[...].T, preferred_element_type=jnp.float32) + rin_b_ref[...]
    m = jnp.mean(tt, axis=1, keepdims=True)
    var = jnp.mean((tt - m) ** 2, axis=1, keepdims=True)
    tt = (tt - m) / jnp.sqrt(var + 1e-5) * rln_g_ref[...] + rln_b_ref[...]
    tt = _lrelu(tt)

    @pl.when(pl.program_id(0) == 0)
    def _():
        tsum_ref[...] = jnp.zeros_like(tsum_ref)
    tsum_ref[...] += jnp.sum(tt, axis=0, keepdims=True)

    tw_ref[...] = jnp.mean(tt.reshape(nb, _WIN, _CQ), axis=2)

    q3 = qq.reshape(nb, _WIN, _DIM)
    k3 = kk.reshape(nb, _WIN, _DIM)
    v3 = vv.reshape(nb, _WIN, _DIM)
    logits = jax.lax.dot_general(q3, k3, (((2,), (2,)), ((0,), (0,))),
                                 preferred_element_type=jnp.float32)
    a = jax.nn.softmax(logits, axis=-1)
    f3 = jax.lax.dot_general(a, v3, (((2,), (1,)), ((0,), (0,))),
                             preferred_element_type=jnp.float32)
    vw_ref[...] = (v3.reshape(nb, _WS, _WS, _DIM)
                   .transpose(1, 0, 2, 3).reshape(_WS, _W, _DIM))
    fw_ref[...] = (f3.reshape(nb, _WS, _WS, _DIM)
                   .transpose(1, 0, 2, 3).reshape(_WS, _W, _DIM))


def _router_kernel(tw_ref, u_ref, tsum_ref,
                   rm1_w_ref, rm1_b_ref, rm2_w_ref, rm2_b_ref,
                   rca_w_ref, rca_b_ref,
                   keep_ref, ca_ref, score_sc):
    tw = tw_ref[...]
    h1 = _lrelu(jnp.dot(tw, rm1_w_ref[...].T, preferred_element_type=jnp.float32)
                + rm1_b_ref[...])
    lg = jnp.dot(h1, rm2_w_ref[...].T, preferred_element_type=jnp.float32) + rm2_b_ref[...]
    pred = jax.nn.softmax(lg, axis=-1)

    g = pred - jnp.log(-jnp.log(u_ref[...]))
    hard = (g[:, 0:1] >= g[:, 1:2]).astype(jnp.float32)
    kcnt = jnp.sum(hard, axis=0, keepdims=True)
    r = kcnt / float(_NW)
    ts = r * _SIGMA_C
    r_hi = ts - (ts - r)
    r_lo = r - r_hi
    diff = (_MF * r_hi - kcnt) + _MF * r_lo
    nk = jnp.clip(kcnt - (diff < 0).astype(jnp.float32), 1.0, float(_NW))

    score = pred[:, 0:1]
    score_sc[...] = score
    st = jnp.transpose(score)
    jl = jax.lax.broadcasted_iota(jnp.int32, (1, _NW), 1)

    def body(ci, carry):
        base = ci * 128
        sc = score_sc[pl.ds(base, 128), :]
        ir = jax.lax.broadcasted_iota(jnp.int32, (128, 1), 0) + base
        gt = st > sc
        tie = (st == sc) & (jl < ir)
        cnt = jnp.sum((gt | tie).astype(jnp.float32), axis=1, keepdims=True)
        keep_ref[pl.ds(base, 128), :] = (cnt < nk).astype(jnp.float32)
        return carry

    jax.lax.fori_loop(0, _NW // 128, body, 0)

    tmean = tsum_ref[...] * (1.0 / _NPIX)
    ca_ref[...] = jax.nn.sigmoid(
        jnp.dot(tmean, rca_w_ref[...].T, preferred_element_type=jnp.float32)
        + rca_b_ref[...])


def _shift_cols(x, s):
    if s == 0:
        return x
    r, c, ch = x.shape
    z = jnp.zeros((r, abs(s), ch), x.dtype)
    if s > 0:
        return jnp.concatenate([x[:, s:, :], z], axis=1)
    return jnp.concatenate([z, x[:, :s, :]], axis=1)


def _stage2_kernel(vp_ref, vc_ref, vn_ref, fp_ref, fc_ref, fn_ref,
                   keep_ref, exp_ref, ca_ref,
                   bdg_ref, ls_b_ref, w1_ref, b1_ref, w2_ref, b2_ref,
                   po_w_ref, po_b_ref, out_ref):
    i = pl.program_id(0)
    mtop = (i > 0).astype(jnp.float32)
    mbot = (i < (_H // _R) - 1).astype(jnp.float32)
    v_ext = jnp.concatenate(
        [vp_ref[...] * mtop, vc_ref[...], vn_ref[...] * mbot], axis=0)
    f_ext = jnp.concatenate(
        [fp_ref[...] * mtop, fc_ref[...], fn_ref[...] * mbot], axis=0)[5:5 + _R + 6]

    ne = _R + 6
    acc = None
    for t in range(9):
        dy, dx = t // 3 - 1, t % 3 - 1
        sl = _shift_cols(v_ext[5 + dy:5 + ne + dy], dx).reshape(ne * _W, _DIM)
        p = jnp.dot(sl, bdg_ref[t], preferred_element_type=jnp.float32)
        acc = p if acc is None else acc + p
    vs = (acc + ls_b_ref[...]).reshape(ne, _W, _DIM)

    gr = jax.lax.broadcasted_iota(jnp.int32, (ne, 1), 0) + (_R * i - 3)
    wr = jnp.clip(gr // _WS, 0, _HH - 1)
    oh = (jax.lax.broadcasted_iota(jnp.int32, (ne, _HH), 1) == wr).astype(jnp.float32)
    krows = jnp.dot(oh, keep_ref[...], preferred_element_type=jnp.float32)
    kpix = jnp.dot(krows, exp_ref[...], preferred_element_type=jnp.float32)
    rmask = ((gr >= 0) & (gr < _H)).astype(jnp.float32)
    ao = jnp.where(kpix[:, :, None] > 0.5, f_ext, vs) * rmask[:, :, None]

    nd = _R + 4
    d1 = None
    for t in range(9):
        dy, dx = t // 3 - 1, t % 3 - 1
        sl = _shift_cols(ao[1 + dy:1 + nd + dy], dx) * w1_ref[t:t + 1, :][None]
        d1 = sl if d1 is None else d1 + sl
    d1 = d1 + b1_ref[...][None]

    sp = None
    for t in range(9):
        dy, dx = (t // 3 - 1) * 2, (t % 3 - 1) * 2
        sl = _shift_cols(d1[2 + dy:2 + _R + dy], dx) * w2_ref[t:t + 1, :][None]
        sp = sl if sp is None else sp + sl
    sp = sp + b2_ref[...][None]

    ge = 0.5 * sp * (1.0 + jax.lax.erf(sp * (2.0 ** -0.5)))
    outp = ge * ca_ref[...][None] + ao[3:3 + _R]
    fin = (jnp.dot(outp.reshape(_R * _W, _DIM), po_w_ref[...].T,
                   preferred_element_type=jnp.float32) + po_b_ref[...])
    out_ref[...] = fin.reshape(_R, _W, _DIM)


def _row2(v):
    return v.reshape(1, -1)


def kernel(x, condition_global, pv_w, pv_b, pq_w, pq_b, pk_w, pk_b,
           cs1_w, cs1_b, cs2_w, cs2_b, po_w, po_b, rin_w, rin_b,
           rln_g, rln_b, rm1_w, rm1_b, rm2_w, rm2_b, rca_w, rca_b,
           ls_w, ls_b):
    f32 = jnp.float32

    xh = x[0].transpose(1, 2, 0)
    cgh = condition_global[0].transpose(1, 2, 0)

    lin = jnp.linspace(-1.0, 1.0, _WS)
    gy, gx = jnp.meshgrid(lin, lin, indexing='ij')
    cwt = jnp.stack([gy, gx], axis=-1).reshape(_WIN, 2).astype(f32)

    u = jax.random.uniform(jax.random.key(42), (1, _NW, 2),
                           minval=1e-6, maxval=1.0 - 1e-6)[0]

    g1 = _H // _R

    sds = jax.ShapeDtypeStruct
    vw, fw, twin, tsum = pl.pallas_call(
        _stage1_kernel,
        grid=(_NW // _WBLK,),
        in_specs=[
            pl.BlockSpec((_WS, _W, _DIM), lambda i: (i, 0, 0)),
            pl.BlockSpec((_WS, _W, 2), lambda i: (i, 0, 0)),
            pl.BlockSpec((_WIN, 2), lambda i: (0, 0)),
            pl.BlockSpec((_DIM, _DIM), lambda i: (0, 0)),
            pl.BlockSpec((1, _DIM), lambda i: (0, 0)),
            pl.BlockSpec((_DIM, _DIM), lambda i: (0, 0)),
            pl.BlockSpec((1, _DIM), lambda i: (0, 0)),
            pl.BlockSpec((_DIM, _DIM), lambda i: (0, 0)),
            pl.BlockSpec((1, _DIM), lambda i: (0, 0)),
            pl.BlockSpec((_CQ, _DIM + 4), lambda i: (0, 0)),
            pl.BlockSpec((1, _CQ), lambda i: (0, 0)),
            pl.BlockSpec((1, _CQ), lambda i: (0, 0)),
            pl.BlockSpec((1, _CQ), lambda i: (0, 0)),
        ],
        out_specs=[
            pl.BlockSpec((_WS, _W, _DIM), lambda i: (i, 0, 0)),
            pl.BlockSpec((_WS, _W, _DIM), lambda i: (i, 0, 0)),
            pl.BlockSpec((_WBLK, _WIN), lambda i: (i, 0)),
            pl.BlockSpec((1, _CQ), lambda i: (0, 0)),
        ],
        out_shape=[
            sds((_H, _W, _DIM), f32),
            sds((_H, _W, _DIM), f32),
            sds((_NW, _WIN), f32),
            sds((1, _CQ), f32),
        ],
    )(xh, cgh, cwt, pv_w, _row2(pv_b), pq_w, _row2(pq_b), pk_w, _row2(pk_b),
      rin_w, _row2(rin_b), _row2(rln_g), _row2(rln_b))

    keep, ca = pl.pallas_call(
        _router_kernel,
        out_shape=[sds((_NW, 1), f32), sds((1, _DIM), f32)],
        scratch_shapes=[pltpu.VMEM((_NW, 1), f32)],
    )(twin, u, tsum, rm1_w, _row2(rm1_b), rm2_w, _row2(rm2_b),
      rca_w, _row2(rca_b))

    vsp, fsp = vw, fw
    keep2d = keep.reshape(_HH, _WW)

    grp = _DIM // 6
    ls9 = jnp.tile(ls_w.transpose(2, 3, 1, 0).reshape(9, grp, _DIM), (1, 6, 1))
    ci = jax.lax.broadcasted_iota(jnp.int32, (_DIM, _DIM), 0) // grp
    co = jax.lax.broadcasted_iota(jnp.int32, (_DIM, _DIM), 1) // grp
    bdg = ls9 * (ci == co).astype(f32)[None]

    pc = jax.lax.broadcasted_iota(jnp.int32, (_WW, _W), 1) // _WS
    wc = jax.lax.broadcasted_iota(jnp.int32, (_WW, _W), 0)
    expm = (pc == wc).astype(f32)

    w1 = cs1_w.reshape(_DIM, 9).T
    w2 = cs2_w.reshape(_DIM, 9).T

    out_sp = pl.pallas_call(
        _stage2_kernel,
        grid=(g1,),
        in_specs=[
            pl.BlockSpec((8, _W, _DIM), lambda i: (jnp.maximum(2 * i - 1, 0), 0, 0)),
            pl.BlockSpec((_R, _W, _DIM), lambda i: (i, 0, 0)),
            pl.BlockSpec((8, _W, _DIM), lambda i: (jnp.minimum(2 * i + 2, _H // 8 - 1), 0, 0)),
            pl.BlockSpec((8, _W, _DIM), lambda i: (jnp.maximum(2 * i - 1, 0), 0, 0)),
            pl.BlockSpec((_R, _W, _DIM), lambda i: (i, 0, 0)),
            pl.BlockSpec((8, _W, _DIM), lambda i: (jnp.minimum(2 * i + 2, _H // 8 - 1), 0, 0)),
            pl.BlockSpec((_HH, _WW), lambda i: (0, 0)),
            pl.BlockSpec((_WW, _W), lambda i: (0, 0)),
            pl.BlockSpec((1, _DIM), lambda i: (0, 0)),
            pl.BlockSpec((9, _DIM, _DIM), lambda i: (0, 0, 0)),
            pl.BlockSpec((1, _DIM), lambda i: (0, 0)),
            pl.BlockSpec((9, _DIM), lambda i: (0, 0)),
            pl.BlockSpec((1, _DIM), lambda i: (0, 0)),
            pl.BlockSpec((9, _DIM), lambda i: (0, 0)),
            pl.BlockSpec((1, _DIM), lambda i: (0, 0)),
            pl.BlockSpec((_DIM, _DIM), lambda i: (0, 0)),
            pl.BlockSpec((1, _DIM), lambda i: (0, 0)),
        ],
        out_specs=[pl.BlockSpec((_R, _W, _DIM), lambda i: (i, 0, 0))],
        out_shape=[sds((_H, _W, _DIM), f32)],
    )(vsp, vsp, vsp, fsp, fsp, fsp, keep2d, expm, ca,
      bdg, _row2(ls_b), w1, _row2(cs1_b), w2, _row2(cs2_b),
      po_w, _row2(po_b))[0]

    return out_sp.transpose(2, 0, 1)[None]
